# baseline (speedup 1.0000x reference)
import functools
import math

import jax
import jax.numpy as jnp
from jax import lax
from jax.experimental import pallas as pl
from jax.experimental.pallas import tpu as pltpu
from jax.experimental.pallas import tpu_sc as plsc

D_MODEL = 64
SCALE = math.sqrt(D_MODEL)
NUM_CORES = 2
NUM_SUBCORES = 16
NUM_WORKERS = NUM_CORES * NUM_SUBCORES
LANES = 16

CHUNK = 128


def _emb_kernel(b_per_w, n_chunks, x_hbm, lut_hbm, out_hbm, idx_v, rows_v, sem):
    wid = lax.axis_index("s") * NUM_CORES + lax.axis_index("c")
    base = wid * b_per_w

    def chunk_body(g, carry):
        off = base + g * CHUNK
        pltpu.sync_copy(x_hbm.at[pl.ds(off, CHUNK)], idx_v)
        pltpu.async_copy(lut_hbm.at[idx_v], rows_v, sem).wait()

        def row_body(r, c):
            for j in range(D_MODEL // LANES):
                sl = pl.ds(j * LANES, LANES)
                rows_v[r, sl] = rows_v[r, sl] * SCALE
            return c

        lax.fori_loop(0, CHUNK, row_body, 0)
        pltpu.sync_copy(rows_v, out_hbm.at[pl.ds(off, CHUNK)])
        return carry

    lax.fori_loop(0, n_chunks, chunk_body, 0)


def kernel(x, lut):
    B = x.shape[0] * x.shape[1]
    xf = x.reshape(B)
    b_per_w = B // NUM_WORKERS
    n_chunks = b_per_w // CHUNK

    mesh = plsc.VectorSubcoreMesh(
        core_axis_name="c",
        subcore_axis_name="s",
        num_cores=NUM_CORES,
        num_subcores=NUM_SUBCORES,
    )

    emb = functools.partial(
        pl.kernel,
        out_type=jax.ShapeDtypeStruct((B, D_MODEL), jnp.float32),
        mesh=mesh,
        scratch_types=[
            pltpu.VMEM((CHUNK,), jnp.int32),
            pltpu.VMEM((CHUNK, D_MODEL), jnp.float32),
            pltpu.SemaphoreType.DMA,
        ],
        compiler_params=pltpu.CompilerParams(use_tc_tiling_on_sc=False),
    )(functools.partial(_emb_kernel, b_per_w, n_chunks))

    out = emb(xf, lut)
    return out.reshape(x.shape[0], x.shape[1], D_MODEL)


# speedup vs baseline: 1.2756x; 1.2756x over previous
import functools
import math

import jax
import jax.numpy as jnp
from jax import lax
from jax.experimental import pallas as pl
from jax.experimental.pallas import tpu as pltpu
from jax.experimental.pallas import tpu_sc as plsc

D_MODEL = 64
SCALE = math.sqrt(D_MODEL)
NUM_CORES = 2
NUM_SUBCORES = 16
NUM_WORKERS = NUM_CORES * NUM_SUBCORES
LANES = 16

IDXW = 128
SUB = 4
CHUNK = IDXW * SUB


def _emb_body(n_chunks, xr_hbm, lut_hbm, out_hbm,
              idx0, idx1, rows0, rows1,
              isem0, isem1, gsem0, gsem1, wsem0, wsem1):
    wid = lax.axis_index("s") * NUM_CORES + lax.axis_index("c")
    base = wid * (n_chunks * CHUNK)
    ibase = wid * (n_chunks * SUB)

    idx = (idx0, idx1)
    rows = (rows0, rows1)
    isem = (isem0, isem1)
    gsem = (gsem0, gsem1)
    wsem = (wsem0, wsem1)

    def idx_fetch(g, b):
        pltpu.async_copy(xr_hbm.at[pl.ds(ibase + g * SUB, SUB)], idx[b], isem[b])

    def idx_wait(b):
        pltpu.make_async_copy(xr_hbm.at[pl.ds(ibase, SUB)], idx[b], isem[b]).wait()

    def gather(b):
        for j in range(SUB):
            pltpu.async_copy(
                lut_hbm.at[idx[b].at[j]],
                rows[b].at[pl.ds(j * IDXW, IDXW)],
                gsem[b],
            )

    def gather_wait(b):
        for j in range(SUB):
            pltpu.make_async_copy(
                lut_hbm.at[idx[b].at[j]],
                rows[b].at[pl.ds(j * IDXW, IDXW)],
                gsem[b],
            ).wait()

    def scale(b):
        r = rows[b]

        @plsc.parallel_loop(0, CHUNK, unroll=4)
        def _(i):
            for j in range(D_MODEL // LANES):
                sl = pl.ds(j * LANES, LANES)
                r[i, sl] = r[i, sl] * SCALE

    def write(g, b):
        pltpu.async_copy(rows[b], out_hbm.at[pl.ds(base + g * CHUNK, CHUNK)], wsem[b])

    def write_wait(b):
        pltpu.make_async_copy(rows[b], out_hbm.at[pl.ds(base, CHUNK)], wsem[b]).wait()

    def step(g, b, first, last2, last):
        nb = 1 - b
        if not last:
            idx_wait(nb)
            if not first:
                write_wait(nb)
            gather(nb)
        gather_wait(b)
        if not last2:
            idx_fetch(g + 2, b)
        scale(b)
        write(g, b)

    idx_fetch(0, 0)
    idx_wait(0)
    gather(0)
    idx_fetch(1, 1)

    step(0, 0, first=True, last2=False, last=False)
    step(1, 1, first=False, last2=False, last=False)

    def pair(p, c):
        g = 2 * p
        step(g, 0, first=False, last2=False, last=False)
        step(g + 1, 1, first=False, last2=False, last=False)
        return c

    lax.fori_loop(1, n_chunks // 2 - 1, pair, 0)

    step(n_chunks - 2, 0, first=False, last2=True, last=False)
    step(n_chunks - 1, 1, first=False, last2=True, last=True)

    write_wait(0)
    write_wait(1)


def kernel(x, lut):
    B = x.shape[0] * x.shape[1]
    xr = x.reshape(B // IDXW, IDXW)
    b_per_w = B // NUM_WORKERS
    n_chunks = b_per_w // CHUNK

    mesh = plsc.VectorSubcoreMesh(
        core_axis_name="c",
        subcore_axis_name="s",
        num_cores=NUM_CORES,
        num_subcores=NUM_SUBCORES,
    )

    emb = functools.partial(
        pl.kernel,
        out_type=jax.ShapeDtypeStruct((B, D_MODEL), jnp.float32),
        mesh=mesh,
        scratch_types=[
            pltpu.VMEM((SUB, IDXW), jnp.int32),
            pltpu.VMEM((SUB, IDXW), jnp.int32),
            pltpu.VMEM((CHUNK, D_MODEL), jnp.float32),
            pltpu.VMEM((CHUNK, D_MODEL), jnp.float32),
            pltpu.SemaphoreType.DMA,
            pltpu.SemaphoreType.DMA,
            pltpu.SemaphoreType.DMA,
            pltpu.SemaphoreType.DMA,
            pltpu.SemaphoreType.DMA,
            pltpu.SemaphoreType.DMA,
        ],
        compiler_params=pltpu.CompilerParams(use_tc_tiling_on_sc=False),
    )(functools.partial(_emb_body, n_chunks))

    out = emb(xr, lut)
    return out.reshape(x.shape[0], x.shape[1], D_MODEL)
